# baseline (speedup 1.0000x reference)
import functools

import numpy as np
import jax
import jax.numpy as jnp
from jax import lax
from jax.experimental import pallas as pl
from jax.experimental.pallas import tpu as pltpu
from jax.experimental.pallas import tpu_sc as plsc

_B, _N, _C = 8, 8192, 16
_S, _NS = 1024, 32
_R2 = np.float32(np.float64(0.2) ** 2)
_NW = 32
_CPT = _B * _S // _NW
_RPT = _CPT * _NS
_DA = 11
_DB = 8
_CA = _DA - 3
_RCHUNK = 512
_K = 16


def _fps_body(xt_ref, nxt_ref, dists_ref):
    x = xt_ref[0]
    y = xt_ref[1]
    z = xt_ref[2]
    xcat = xt_ref[...].reshape(3 * _B, _N)
    lane = lax.broadcasted_iota(jnp.int32, (_B, _N), 1)
    lane_s = lax.broadcasted_iota(jnp.int32, (_B, _S), 1)
    lane24 = lax.broadcasted_iota(jnp.int32, (_B, 3 * _B), 1)
    row24 = lax.broadcasted_iota(jnp.int32, (_B, 3 * _B), 0)
    nxt_ref[...] = jnp.zeros((3, _B, _S), jnp.float32)
    dists_ref[...] = jnp.full((_B, _N), 1e10, jnp.float32)

    def body(i, far):
        sel = lane == far
        selc = jnp.where(sel, 1.0, 0.0)
        cm = lax.dot_general(selc, xcat, (((1,), (1,)), ((), ())),
                             preferred_element_type=jnp.float32,
                             precision=lax.Precision.HIGHEST)
        cx = jnp.max(jnp.where(lane24 == row24, cm, -1.0),
                     axis=1, keepdims=True)
        cy = jnp.max(jnp.where(lane24 == row24 + _B, cm, -1.0),
                     axis=1, keepdims=True)
        cz = jnp.max(jnp.where(lane24 == row24 + 2 * _B, cm, -1.0),
                     axis=1, keepdims=True)
        hit = lane_s == i
        nxt_ref[0] = jnp.where(hit, cx, nxt_ref[0])
        nxt_ref[1] = jnp.where(hit, cy, nxt_ref[1])
        nxt_ref[2] = jnp.where(hit, cz, nxt_ref[2])
        dx = x - cx
        dy = y - cy
        dz = z - cz
        d = dx * dx + dy * dy + dz * dz
        dn = jnp.minimum(dists_ref[...], d)
        dists_ref[...] = dn
        m = jnp.max(dn, axis=1, keepdims=True)
        return jnp.min(jnp.where(dn == m, lane, _N), axis=1, keepdims=True)

    lax.fori_loop(0, _S, body, jnp.zeros((_B, 1), jnp.int32), unroll=2)


_fps = pl.pallas_call(
    _fps_body,
    out_shape=jax.ShapeDtypeStruct((3, _B, _S), jnp.float32),
    scratch_shapes=[pltpu.VMEM((_B, _N), jnp.float32)],
)


def _group_body(xt_hbm, nxt_hbm, feat_hbm, ga_hbm, gb_hbm, xv, cv, fv, ov, oa, ob):
    wid = lax.axis_index("s") * 2 + lax.axis_index("c")
    b = wid // 4
    q = wid - b * 4
    pltpu.sync_copy(xt_hbm.at[:, b, :], xv)
    pltpu.sync_copy(nxt_hbm.at[:, b, pl.ds(q * _CPT, _CPT)], cv)
    iota16 = lax.iota(jnp.int32, 16)

    def group(g, carry):
        c0 = g * 16
        cx = cv[0, pl.ds(c0, 16)]
        cy = cv[1, pl.ds(c0, 16)]
        cz = cv[2, pl.ds(c0, 16)]
        obase = (c0 + iota16) * _NS

        def scan_body(blk, cnt):
            p = blk * _K
            px16 = xv[0, pl.ds(p, _K)]
            py16 = xv[1, pl.ds(p, _K)]
            pz16 = xv[2, pl.ds(p, _K)]
            for k in range(_K):
                dx = jnp.full((16,), px16[k]) - cx
                dy = jnp.full((16,), py16[k]) - cy
                dz = jnp.full((16,), pz16[k]) - cz
                dd = dx * dx + dy * dy + dz * dz
                hit = jnp.logical_and(dd < _R2, cnt < _NS)
                plsc.store_scatter(ov, [obase + cnt],
                                   jnp.full((16,), p + k, jnp.int32),
                                   mask=hit)
                cnt = cnt + hit.astype(jnp.int32)
            return cnt

        cnt = lax.fori_loop(0, _N // _K, scan_body,
                            jnp.zeros((16,), jnp.int32))
        first = plsc.load_gather(ov, [obase])
        for j in range(1, _NS):
            plsc.store_scatter(ov, [obase + j], first, mask=cnt <= j)
        return carry

    lax.fori_loop(0, _CPT // 16, group, 0)

    def splat(v):
        return jnp.full((16,), v, jnp.int32)

    def assemble(half, fch0, nch, oc, oc_hbm, dw, xyz_part):
        pltpu.sync_copy(feat_hbm.at[b, pl.ds(fch0, nch), :], fv.at[pl.ds(0, nch)])

        def chunk(c, carry):
            r0 = c * _RCHUNK

            def blk(i, carry2):
                rb = i * 16
                idx16 = ov[pl.ds(r0 + rb, 16)]
                s = (r0 + rb) // _NS
                ch = 0
                if xyz_part:
                    for cc in range(3):
                        v = (plsc.load_gather(xv, [splat(cc), idx16])
                             - plsc.load_gather(cv, [splat(cc), splat(s)]))
                        oc[ch, pl.ds(rb, 16)] = v
                        ch += 1
                for cc in range(nch):
                    v = plsc.load_gather(fv, [splat(cc), idx16])
                    oc[ch, pl.ds(rb, 16)] = v
                    ch += 1
                return carry2

            lax.fori_loop(0, _RCHUNK // 16, blk, 0)
            for cc in range(dw):
                pltpu.sync_copy(
                    oc.at[pl.ds(cc, 1)],
                    oc_hbm.at[pl.ds(cc, 1), pl.ds(wid * _RPT + r0, _RCHUNK)])
            return carry

        lax.fori_loop(0, _RPT // _RCHUNK, chunk, 0)

    assemble(0, 0, _CA, oa, ga_hbm, _DA, True)
    assemble(1, _CA, _C - _CA, ob, gb_hbm, _DB, False)


@functools.cache
def _sc_group_kernel():
    mesh = plsc.VectorSubcoreMesh(core_axis_name="c", subcore_axis_name="s")
    return functools.partial(
        pl.kernel,
        mesh=mesh,
        compiler_params=pltpu.CompilerParams(needs_layout_passes=False),
        out_type=[
            jax.ShapeDtypeStruct((_DA, _B * _S * _NS), jnp.float32),
            jax.ShapeDtypeStruct((_DB, _B * _S * _NS), jnp.float32),
        ],
        scratch_types=[
            pltpu.VMEM((3, _N), jnp.float32),
            pltpu.VMEM((3, _CPT), jnp.float32),
            pltpu.VMEM((_CA, _N), jnp.float32),
            pltpu.VMEM((_RPT,), jnp.int32),
            pltpu.VMEM((_DA, _RCHUNK), jnp.float32),
            pltpu.VMEM((_DB, _RCHUNK), jnp.float32),
        ],
    )(_group_body)


def _dot_l(w, x):
    return lax.dot_general(w, x, (((1,), (0,)), ((), ())),
                           preferred_element_type=jnp.float32,
                           precision=lax.Precision.HIGHEST)


def _mlp_body(ga_ref, gb_ref, nxt_ref, w0a_ref, w0b_ref, b0_ref, w1_ref,
              b1_ref, w2_ref, b2_ref, feat_ref, nxout_ref):
    blk = _S // 4
    xa = ga_ref[...]
    xb = gb_ref[...]
    h = _dot_l(w0a_ref[...], xa) + _dot_l(w0b_ref[...], xb)
    h = jnp.maximum(h + b0_ref[...], 0.0)
    h = jnp.maximum(_dot_l(w1_ref[...], h) + b1_ref[...], 0.0)
    h = jnp.maximum(_dot_l(w2_ref[...], h) + b2_ref[...], 0.0)
    m = jnp.max(jnp.reshape(h, (64, blk, _NS)), axis=2)
    feat_ref[0] = m
    nxout_ref[0] = nxt_ref[0].T


_mlp_in_specs = [
    pl.BlockSpec((_DA, _S // 4 * _NS), lambda b, q: (0, b * 4 + q)),
    pl.BlockSpec((_DB, _S // 4 * _NS), lambda b, q: (0, b * 4 + q)),
    pl.BlockSpec((1, 3, _S // 4), lambda b, q: (b, 0, q)),
    pl.BlockSpec((32, _DA), lambda b, q: (0, 0)),
    pl.BlockSpec((32, _DB), lambda b, q: (0, 0)),
    pl.BlockSpec((32, 1), lambda b, q: (0, 0)),
    pl.BlockSpec((32, 32), lambda b, q: (0, 0)),
    pl.BlockSpec((32, 1), lambda b, q: (0, 0)),
    pl.BlockSpec((64, 32), lambda b, q: (0, 0)),
    pl.BlockSpec((64, 1), lambda b, q: (0, 0)),
]
_mlp_out_specs = [
    pl.BlockSpec((1, 64, _S // 4), lambda b, q: (b, 0, q)),
    pl.BlockSpec((1, _S // 4, 3), lambda b, q: (b, q, 0)),
]
_mlp = pl.pallas_call(
    _mlp_body,
    grid=(_B, 4),
    in_specs=_mlp_in_specs,
    out_specs=_mlp_out_specs,
    out_shape=[
        jax.ShapeDtypeStruct((_B, 64, _S), jnp.float32),
        jax.ShapeDtypeStruct((_B, _S, 3), jnp.float32),
    ],
)


def kernel(xyz, features, W0, b0, W1, b1, W2, b2):
    grp = _sc_group_kernel()
    xt = jnp.transpose(xyz, (2, 0, 1))
    nxt = _fps(xt)
    ga, gb = grp(xt, nxt, features)
    feat, nxout = _mlp(ga, gb, jnp.transpose(nxt, (1, 0, 2)),
                       W0[:, :_DA], W0[:, _DA:], b0.reshape(32, 1),
                       W1, b1.reshape(32, 1), W2, b2.reshape(64, 1))
    return nxout, feat


# speedup vs baseline: 2.0107x; 2.0107x over previous
import functools

import numpy as np
import jax
import jax.numpy as jnp
from jax import lax
from jax.experimental import pallas as pl
from jax.experimental.pallas import tpu as pltpu
from jax.experimental.pallas import tpu_sc as plsc

_B, _N, _C = 8, 8192, 16
_S, _NS = 1024, 32
_R2 = np.float32(np.float64(0.2) ** 2)
_NW = 32
_CPT = _B * _S // _NW
_RPT = _CPT * _NS
_DA = 11
_DB = 8
_CA = _DA - 3
_RCHUNK = 512
_K = 16


def _fps_body(xt_ref, nxt_ref, dists_ref):
    x = xt_ref[0]
    y = xt_ref[1]
    z = xt_ref[2]
    lane = lax.broadcasted_iota(jnp.int32, (_B, _N), 1)
    lane_s = lax.broadcasted_iota(jnp.int32, (_B, _S), 1)
    nxt_ref[...] = jnp.zeros((3, _B, _S), jnp.float32)
    dists_ref[...] = jnp.full((_B, _N), 1e10, jnp.float32)

    def body(i, far):
        sel = lane == far
        cx = jnp.max(jnp.where(sel, x, -1.0), axis=1, keepdims=True)
        cy = jnp.max(jnp.where(sel, y, -1.0), axis=1, keepdims=True)
        cz = jnp.max(jnp.where(sel, z, -1.0), axis=1, keepdims=True)
        hit = lane_s == i
        nxt_ref[0] = jnp.where(hit, cx, nxt_ref[0])
        nxt_ref[1] = jnp.where(hit, cy, nxt_ref[1])
        nxt_ref[2] = jnp.where(hit, cz, nxt_ref[2])
        dx = x - cx
        dy = y - cy
        dz = z - cz
        d = dx * dx + dy * dy + dz * dz
        dn = jnp.minimum(dists_ref[...], d)
        dists_ref[...] = dn
        m = jnp.max(dn, axis=1, keepdims=True)
        return jnp.min(jnp.where(dn == m, lane, _N), axis=1, keepdims=True)

    lax.fori_loop(0, _S, body, jnp.zeros((_B, 1), jnp.int32), unroll=2)


_fps = pl.pallas_call(
    _fps_body,
    out_shape=jax.ShapeDtypeStruct((3, _B, _S), jnp.float32),
    scratch_shapes=[pltpu.VMEM((_B, _N), jnp.float32)],
)


def _group_body(xt_hbm, nxt_hbm, feat_hbm, ga_hbm, gb_hbm, xv, cv, fv, ov, oa, ob):
    wid = lax.axis_index("s") * 2 + lax.axis_index("c")
    b = wid // 4
    q = wid - b * 4
    pltpu.sync_copy(xt_hbm.at[:, b, :], xv)
    pltpu.sync_copy(nxt_hbm.at[:, b, pl.ds(q * _CPT, _CPT)], cv)
    iota16 = lax.iota(jnp.int32, 16)

    def group(g, carry):
        c0 = g * 16
        cx = cv[0, pl.ds(c0, 16)]
        cy = cv[1, pl.ds(c0, 16)]
        cz = cv[2, pl.ds(c0, 16)]
        obase = (c0 + iota16) * _NS

        def scan_body(blk, cnt):
            p = blk * _K
            px16 = xv[0, pl.ds(p, _K)]
            py16 = xv[1, pl.ds(p, _K)]
            pz16 = xv[2, pl.ds(p, _K)]
            for k in range(_K):
                dx = jnp.full((16,), px16[k]) - cx
                dy = jnp.full((16,), py16[k]) - cy
                dz = jnp.full((16,), pz16[k]) - cz
                dd = dx * dx + dy * dy + dz * dz
                hit = jnp.logical_and(dd < _R2, cnt < _NS)
                plsc.store_scatter(ov, [obase + cnt],
                                   jnp.full((16,), p + k, jnp.int32),
                                   mask=hit)
                cnt = cnt + hit.astype(jnp.int32)
            return cnt

        cnt = lax.fori_loop(0, _N // _K, scan_body,
                            jnp.zeros((16,), jnp.int32))
        first = plsc.load_gather(ov, [obase])
        for j in range(1, _NS):
            plsc.store_scatter(ov, [obase + j], first, mask=cnt <= j)
        return carry

    lax.fori_loop(0, _CPT // 16, group, 0)

    def splat(v):
        return jnp.full((16,), v, jnp.int32)

    def assemble(half, fch0, nch, oc, oc_hbm, dw, xyz_part):
        pltpu.sync_copy(feat_hbm.at[b, pl.ds(fch0, nch), :], fv.at[pl.ds(0, nch)])

        def chunk(c, carry):
            r0 = c * _RCHUNK

            def blk(i, carry2):
                rb = i * 16
                idx16 = ov[pl.ds(r0 + rb, 16)]
                s = (r0 + rb) // _NS
                ch = 0
                if xyz_part:
                    for cc in range(3):
                        v = (plsc.load_gather(xv, [splat(cc), idx16])
                             - plsc.load_gather(cv, [splat(cc), splat(s)]))
                        oc[ch, pl.ds(rb, 16)] = v
                        ch += 1
                for cc in range(nch):
                    v = plsc.load_gather(fv, [splat(cc), idx16])
                    oc[ch, pl.ds(rb, 16)] = v
                    ch += 1
                return carry2

            lax.fori_loop(0, _RCHUNK // 16, blk, 0)
            for cc in range(dw):
                pltpu.sync_copy(
                    oc.at[pl.ds(cc, 1)],
                    oc_hbm.at[pl.ds(cc, 1), pl.ds(wid * _RPT + r0, _RCHUNK)])
            return carry

        lax.fori_loop(0, _RPT // _RCHUNK, chunk, 0)

    assemble(0, 0, _CA, oa, ga_hbm, _DA, True)
    assemble(1, _CA, _C - _CA, ob, gb_hbm, _DB, False)


@functools.cache
def _sc_group_kernel():
    mesh = plsc.VectorSubcoreMesh(core_axis_name="c", subcore_axis_name="s")
    return functools.partial(
        pl.kernel,
        mesh=mesh,
        compiler_params=pltpu.CompilerParams(needs_layout_passes=False),
        out_type=[
            jax.ShapeDtypeStruct((_DA, _B * _S * _NS), jnp.float32),
            jax.ShapeDtypeStruct((_DB, _B * _S * _NS), jnp.float32),
        ],
        scratch_types=[
            pltpu.VMEM((3, _N), jnp.float32),
            pltpu.VMEM((3, _CPT), jnp.float32),
            pltpu.VMEM((_CA, _N), jnp.float32),
            pltpu.VMEM((_RPT,), jnp.int32),
            pltpu.VMEM((_DA, _RCHUNK), jnp.float32),
            pltpu.VMEM((_DB, _RCHUNK), jnp.float32),
        ],
    )(_group_body)


def _dot_l(w, x):
    return lax.dot_general(w, x, (((1,), (0,)), ((), ())),
                           preferred_element_type=jnp.float32,
                           precision=lax.Precision.HIGHEST)


def _mlp_body(ga_ref, gb_ref, nxt_ref, w0a_ref, w0b_ref, b0_ref, w1_ref,
              b1_ref, w2_ref, b2_ref, feat_ref, nxout_ref):
    blk = _S // 4
    xa = ga_ref[...]
    xb = gb_ref[...]
    h = _dot_l(w0a_ref[...], xa) + _dot_l(w0b_ref[...], xb)
    h = jnp.maximum(h + b0_ref[...], 0.0)
    h = jnp.maximum(_dot_l(w1_ref[...], h) + b1_ref[...], 0.0)
    h = jnp.maximum(_dot_l(w2_ref[...], h) + b2_ref[...], 0.0)
    m = jnp.max(jnp.reshape(h, (64, blk, _NS)), axis=2)
    feat_ref[0] = m
    nxout_ref[0] = nxt_ref[0].T


_mlp_in_specs = [
    pl.BlockSpec((_DA, _S // 4 * _NS), lambda b, q: (0, b * 4 + q)),
    pl.BlockSpec((_DB, _S // 4 * _NS), lambda b, q: (0, b * 4 + q)),
    pl.BlockSpec((1, 3, _S // 4), lambda b, q: (b, 0, q)),
    pl.BlockSpec((32, _DA), lambda b, q: (0, 0)),
    pl.BlockSpec((32, _DB), lambda b, q: (0, 0)),
    pl.BlockSpec((32, 1), lambda b, q: (0, 0)),
    pl.BlockSpec((32, 32), lambda b, q: (0, 0)),
    pl.BlockSpec((32, 1), lambda b, q: (0, 0)),
    pl.BlockSpec((64, 32), lambda b, q: (0, 0)),
    pl.BlockSpec((64, 1), lambda b, q: (0, 0)),
]
_mlp_out_specs = [
    pl.BlockSpec((1, 64, _S // 4), lambda b, q: (b, 0, q)),
    pl.BlockSpec((1, _S // 4, 3), lambda b, q: (b, q, 0)),
]
_mlp = pl.pallas_call(
    _mlp_body,
    grid=(_B, 4),
    in_specs=_mlp_in_specs,
    out_specs=_mlp_out_specs,
    out_shape=[
        jax.ShapeDtypeStruct((_B, 64, _S), jnp.float32),
        jax.ShapeDtypeStruct((_B, _S, 3), jnp.float32),
    ],
)


def kernel(xyz, features, W0, b0, W1, b1, W2, b2):
    grp = _sc_group_kernel()
    xt = jnp.transpose(xyz, (2, 0, 1))
    nxt = _fps(xt)
    ga, gb = grp(xt, nxt, features)
    feat, nxout = _mlp(ga, gb, jnp.transpose(nxt, (1, 0, 2)),
                       W0[:, :_DA], W0[:, _DA:], b0.reshape(32, 1),
                       W1, b1.reshape(32, 1), W2, b2.reshape(64, 1))
    return nxout, feat
